# Initial kernel scaffold; baseline (speedup 1.0000x reference)
import jax
import jax.numpy as jnp
from jax.experimental import pallas as pl


def kernel(g, x_t, time_embed, params):
    raise NotImplementedError("write your pallas kernel here")


# speedup vs baseline: 3.4422x; 3.4422x over previous
import jax
import jax.numpy as jnp
from jax.experimental import pallas as pl
from jax.experimental.pallas import tpu as pltpu

N = 10000
H = 128
NUM_LAYERS = 2
ROWS = 1000


def _gelu(z):
    return 0.5 * z * (1.0 + jax.lax.erf(z * 0.7071067811865476))


def _layer_norm(z, g, b, eps=1e-5):
    mu = jnp.mean(z, axis=-1, keepdims=True)
    var = jnp.mean((z - mu) ** 2, axis=-1, keepdims=True)
    return (z - mu) * jax.lax.rsqrt(var + eps) * g + b


def _unet_body(x_ref, t_ref, *refs):
    *w_refs, out_ref, hout_ref = refs
    it = iter(w_refs)

    def nxt():
        return next(it)[...]

    def dot(a, b):
        return jnp.dot(a, b, preferred_element_type=jnp.float32)

    h = _gelu(dot(x_ref[...], nxt()) + dot(t_ref[...], nxt()) + nxt())

    skips = []
    for _ in range(NUM_LAYERS):
        skips.append(h)
        z = dot(h, nxt()) + nxt()
        z = _layer_norm(z, nxt(), nxt())
        z = _gelu(z)
        z = dot(z, nxt()) + nxt()
        h = z + h

    z = dot(h, nxt()) + nxt()
    z = _layer_norm(z, nxt(), nxt())
    z = _gelu(z)
    z = dot(z, nxt()) + nxt()
    h = z + h

    for i in range(NUM_LAYERS):
        skip = skips[NUM_LAYERS - 1 - i]
        z = dot(h, nxt()) + dot(skip, nxt()) + nxt()
        z = _layer_norm(z, nxt(), nxt())
        z = _gelu(z)
        z = dot(z, nxt()) + nxt()
        h = z

    out_ref[...] = dot(h, nxt()) + nxt()
    hout_ref[...] = h


def kernel(g, x_t, time_embed, params):
    del g
    p = params
    vec = lambda v: v.reshape(1, H)
    ws = [p['init_w'][:H], p['init_w'][H:], vec(p['init_b'])]
    for i in range(NUM_LAYERS):
        ws += [p[f'down{i}_w1'], vec(p[f'down{i}_b1']),
               vec(p[f'down{i}_g']), vec(p[f'down{i}_be']),
               p[f'down{i}_w2'], vec(p[f'down{i}_b2'])]
    ws += [p['mid_w1'], vec(p['mid_b1']), vec(p['mid_g']), vec(p['mid_be']),
           p['mid_w2'], vec(p['mid_b2'])]
    for i in range(NUM_LAYERS):
        ws += [p[f'up{i}_w1'][:H], p[f'up{i}_w1'][H:], vec(p[f'up{i}_b1']),
               vec(p[f'up{i}_g']), vec(p[f'up{i}_be']),
               p[f'up{i}_w2'], vec(p[f'up{i}_b2'])]
    ws += [p['final_w'], vec(p['final_b'])]

    grid = N // ROWS
    row_spec = pl.BlockSpec((ROWS, H), lambda i: (i, 0))
    w_specs = [pl.BlockSpec(w.shape, lambda i: (0, 0)) for w in ws]

    out, h = pl.pallas_call(
        _unet_body,
        grid=(grid,),
        in_specs=[row_spec, row_spec] + w_specs,
        out_specs=[row_spec, row_spec],
        out_shape=[jax.ShapeDtypeStruct((N, H), jnp.float32),
                   jax.ShapeDtypeStruct((N, H), jnp.float32)],
        compiler_params=pltpu.CompilerParams(
            dimension_semantics=("arbitrary",)),
    )(x_t, time_embed, *ws)
    return (out, h)


# speedup vs baseline: 4.4363x; 1.2888x over previous
import jax
import jax.numpy as jnp
from jax.experimental import pallas as pl
from jax.experimental.pallas import tpu as pltpu

N = 10000
H = 128
NUM_LAYERS = 2
ROWS = 2000


def _gelu(z):
    return 0.5 * z * (1.0 + jax.lax.erf(z * 0.7071067811865476))


def _layer_norm(z, g, b, eps=1e-5):
    mu = jnp.mean(z, axis=-1, keepdims=True)
    var = jnp.mean((z - mu) ** 2, axis=-1, keepdims=True)
    return (z - mu) * jax.lax.rsqrt(var + eps) * g + b


def _unet_body(x_ref, t_ref, *refs):
    *w_refs, out_ref, hout_ref = refs
    it = iter(w_refs)

    def nxt():
        return next(it)[...]

    def dot(a, b):
        return jnp.dot(a, b, preferred_element_type=jnp.float32)

    h = _gelu(dot(x_ref[...], nxt()) + dot(t_ref[...], nxt()) + nxt())

    skips = []
    for _ in range(NUM_LAYERS):
        skips.append(h)
        z = dot(h, nxt()) + nxt()
        z = _layer_norm(z, nxt(), nxt())
        z = _gelu(z)
        z = dot(z, nxt()) + nxt()
        h = z + h

    z = dot(h, nxt()) + nxt()
    z = _layer_norm(z, nxt(), nxt())
    z = _gelu(z)
    z = dot(z, nxt()) + nxt()
    h = z + h

    for i in range(NUM_LAYERS):
        skip = skips[NUM_LAYERS - 1 - i]
        z = dot(h, nxt()) + dot(skip, nxt()) + nxt()
        z = _layer_norm(z, nxt(), nxt())
        z = _gelu(z)
        z = dot(z, nxt()) + nxt()
        h = z

    out_ref[...] = dot(h, nxt()) + nxt()
    hout_ref[...] = h


def kernel(g, x_t, time_embed, params):
    del g
    p = params
    vec = lambda v: v.reshape(1, H)
    ws = [p['init_w'][:H], p['init_w'][H:], vec(p['init_b'])]
    for i in range(NUM_LAYERS):
        ws += [p[f'down{i}_w1'], vec(p[f'down{i}_b1']),
               vec(p[f'down{i}_g']), vec(p[f'down{i}_be']),
               p[f'down{i}_w2'], vec(p[f'down{i}_b2'])]
    ws += [p['mid_w1'], vec(p['mid_b1']), vec(p['mid_g']), vec(p['mid_be']),
           p['mid_w2'], vec(p['mid_b2'])]
    for i in range(NUM_LAYERS):
        ws += [p[f'up{i}_w1'][:H], p[f'up{i}_w1'][H:], vec(p[f'up{i}_b1']),
               vec(p[f'up{i}_g']), vec(p[f'up{i}_be']),
               p[f'up{i}_w2'], vec(p[f'up{i}_b2'])]
    ws += [p['final_w'], vec(p['final_b'])]

    grid = N // ROWS
    row_spec = pl.BlockSpec((ROWS, H), lambda i: (i, 0))
    w_specs = [pl.BlockSpec(w.shape, lambda i: (0, 0)) for w in ws]

    out, h = pl.pallas_call(
        _unet_body,
        grid=(grid,),
        in_specs=[row_spec, row_spec] + w_specs,
        out_specs=[row_spec, row_spec],
        out_shape=[jax.ShapeDtypeStruct((N, H), jnp.float32),
                   jax.ShapeDtypeStruct((N, H), jnp.float32)],
        compiler_params=pltpu.CompilerParams(
            dimension_semantics=("parallel",)),
    )(x_t, time_embed, *ws)
    return (out, h)
